# baseline (speedup 1.0000x reference)
import jax
import jax.numpy as jnp
from jax import lax
from jax.experimental import pallas as pl
from jax.experimental.pallas import tpu as pltpu
from jax.experimental.pallas import tpu_sc as plsc

N = 10000
E = 320000
D = 128

NC = 2
NS = 16

N_PAD = 10240
NPT = N_PAD // NS
PAD_NODE = N_PAD - 1

B = 112
E_PAD = 322560
EDGE_ROWS = E_PAD // B
RPT = EDGE_ROWS // (NC * NS)
NBLK = 6
BLK = RPT // NBLK

ZR_DEG = 64
ZC_DEG = NPT // ZR_DEG

_MESH = plsc.VectorSubcoreMesh(core_axis_name="c", subcore_axis_name="s")
_SC_PARAMS = pltpu.CompilerParams(use_tc_tiling_on_sc=False)


def _zero_vmem(ref, nrows, ncols, dtype=jnp.float32):
    zeros16 = jnp.zeros((16,), dtype)

    def body(i, _):
        for col in range(ncols // 16):
            ref[i, pl.ds(col * 16, 16)] = zeros16
        return 0

    lax.fori_loop(0, nrows, body, 0)


def _deg_body(src_hbm, dst_hbm, out_hbm,
              deg_s, deg_d, src_v, dst_v, ones_v, zbuf, sem_a, sem_b):
    c = lax.axis_index("c")
    s = lax.axis_index("s")

    _zero_vmem(zbuf, ZR_DEG, 16)
    for k in range(ZC_DEG):
        base = s * NPT + k * ZR_DEG
        pltpu.async_copy(zbuf, deg_s.at[pl.ds(base, ZR_DEG)], sem_a)
        pltpu.async_copy(zbuf, deg_d.at[pl.ds(base, ZR_DEG)], sem_b)

    ones16 = jnp.ones((16,), jnp.float32)

    def ones_body(i, _):
        ones_v[i, :] = ones16
        return 0

    lax.fori_loop(0, B, ones_body, 0)

    row0 = c * (EDGE_ROWS // NC) + s * RPT
    pltpu.sync_copy(src_hbm.at[pl.ds(row0, RPT)], src_v)
    pltpu.sync_copy(dst_hbm.at[pl.ds(row0, RPT)], dst_v)

    for k in range(ZC_DEG):
        base = s * NPT + k * ZR_DEG
        pltpu.make_async_copy(zbuf, deg_s.at[pl.ds(base, ZR_DEG)], sem_a).wait()
        pltpu.make_async_copy(zbuf, deg_d.at[pl.ds(base, ZR_DEG)], sem_b).wait()

    plsc.subcore_barrier()

    def chunk(j, _):
        pltpu.async_copy(ones_v, deg_s.at[src_v.at[j]], sem_a, add=True)
        pltpu.async_copy(ones_v, deg_d.at[dst_v.at[j]], sem_b, add=True)

        @pl.when(j >= 4)
        def _():
            pltpu.make_async_copy(ones_v, deg_s.at[src_v.at[j - 4]], sem_a).wait()
            pltpu.make_async_copy(ones_v, deg_d.at[dst_v.at[j - 4]], sem_b).wait()

        return 0

    lax.fori_loop(0, RPT, chunk, 0)
    for j in range(RPT - 4, RPT):
        pltpu.make_async_copy(ones_v, deg_s.at[src_v.at[j]], sem_a).wait()
        pltpu.make_async_copy(ones_v, deg_d.at[dst_v.at[j]], sem_b).wait()

    plsc.subcore_barrier()

    base = s * NPT
    pltpu.sync_copy(deg_s.at[pl.ds(base, NPT)], out_hbm.at[c, 0, pl.ds(base, NPT)])
    pltpu.sync_copy(deg_d.at[pl.ds(base, NPT)], out_hbm.at[c, 1, pl.ds(base, NPT)])


_deg_kernel = pl.kernel(
    _deg_body,
    out_type=jax.ShapeDtypeStruct((NC, 2, N_PAD, 16), jnp.float32),
    mesh=_MESH,
    compiler_params=_SC_PARAMS,
    scratch_types=[
        pltpu.VMEM_SHARED((N_PAD, 16), jnp.float32),
        pltpu.VMEM_SHARED((N_PAD, 16), jnp.float32),
        pltpu.VMEM((RPT, B), jnp.int32),
        pltpu.VMEM((RPT, B), jnp.int32),
        pltpu.VMEM((B, 16), jnp.float32),
        pltpu.VMEM((ZR_DEG, 16), jnp.float32),
        pltpu.SemaphoreType.DMA,
        pltpu.SemaphoreType.DMA,
    ],
)


def _agg_body(h_hbm, src_hbm, dst_hbm, out_hbm,
              acc, src_v, dst_v, rows0, rows1, rows2,
              g0, g1, g2, s0, s1, s2):
    c = lax.axis_index("c")
    s = lax.axis_index("s")

    rows = (rows0, rows1, rows2)
    gsem = (g0, g1, g2)
    ssem = (s0, s1, s2)

    _zero_vmem(rows0, B, D)
    base = s * NPT
    for k in range(5):
        pltpu.async_copy(rows0, acc.at[pl.ds(base + k * B, B)], g0)
    pltpu.async_copy(rows0.at[pl.ds(0, NPT - 5 * B)],
                     acc.at[pl.ds(base + 5 * B, NPT - 5 * B)], g1)

    row0 = c * (EDGE_ROWS // NC) + s * RPT
    pltpu.sync_copy(src_hbm.at[pl.ds(row0, BLK)], src_v)
    pltpu.sync_copy(dst_hbm.at[pl.ds(row0, BLK)], dst_v)

    for k in range(5):
        pltpu.make_async_copy(rows0, acc.at[pl.ds(base + k * B, B)], g0).wait()
    pltpu.make_async_copy(rows0.at[pl.ds(0, NPT - 5 * B)],
                          acc.at[pl.ds(base + 5 * B, NPT - 5 * B)], g1).wait()

    plsc.subcore_barrier()

    for blk in range(NBLK):
        if blk > 0:
            brow = row0 + blk * BLK
            pltpu.sync_copy(src_hbm.at[pl.ds(brow, BLK)], src_v)
            pltpu.sync_copy(dst_hbm.at[pl.ds(brow, BLK)], dst_v)

        pltpu.async_copy(h_hbm.at[src_v.at[0]], rows0, g0)
        pltpu.async_copy(h_hbm.at[src_v.at[1]], rows1, g1)

        def slot(j, b, bp):
            pltpu.make_async_copy(h_hbm.at[src_v.at[j]], rows[b], gsem[b]).wait()
            pltpu.async_copy(rows[b], acc.at[dst_v.at[j]], ssem[b], add=True)

            @pl.when(j >= 1)
            def _():
                pltpu.make_async_copy(rows[bp], acc.at[dst_v.at[j - 1]],
                                      ssem[bp]).wait()

            @pl.when(j + 2 < BLK)
            def _():
                pltpu.async_copy(h_hbm.at[src_v.at[j + 2]], rows[bp], gsem[bp])

        def tri(k, _):
            for i in range(3):
                slot(k * 3 + i, i, (i + 2) % 3)
            return 0

        lax.fori_loop(0, BLK // 3, tri, 0)
        bl = BLK - 1
        pltpu.make_async_copy(rows[bl % 3], acc.at[dst_v.at[bl]],
                              ssem[bl % 3]).wait()

    plsc.subcore_barrier()

    pltpu.sync_copy(acc.at[pl.ds(base, NPT)], out_hbm.at[c, pl.ds(base, NPT)])


_agg_kernel = pl.kernel(
    _agg_body,
    out_type=jax.ShapeDtypeStruct((NC, N_PAD, D), jnp.float32),
    mesh=_MESH,
    compiler_params=_SC_PARAMS,
    scratch_types=[
        pltpu.VMEM_SHARED((N_PAD, D), jnp.float32),
        pltpu.VMEM((BLK, B), jnp.int32),
        pltpu.VMEM((BLK, B), jnp.int32),
        pltpu.VMEM((B, D), jnp.float32),
        pltpu.VMEM((B, D), jnp.float32),
        pltpu.VMEM((B, D), jnp.float32),
        pltpu.SemaphoreType.DMA,
        pltpu.SemaphoreType.DMA,
        pltpu.SemaphoreType.DMA,
        pltpu.SemaphoreType.DMA,
        pltpu.SemaphoreType.DMA,
        pltpu.SemaphoreType.DMA,
    ],
)


RB = 512
GRID = N_PAD // RB


def _norms(deg_ref, rb, which):
    dp = deg_ref[...]
    d_p = dp[0, which] + dp[1, which]
    rq = lax.broadcasted_iota(jnp.int32, (rb, rb // 8), 0) // 8
    qq = lax.broadcasted_iota(jnp.int32, (rb, rb // 8), 1)
    a = (rq == qq).astype(jnp.float32)
    ex = jnp.dot(a, d_p, preferred_element_type=jnp.float32)
    rr = lax.broadcasted_iota(jnp.int32, (rb, 128), 0) % 8
    ll = lax.broadcasted_iota(jnp.int32, (rb, 128), 1)
    sel = (ll == 16 * rr).astype(jnp.float32)
    d_col = jnp.sum(ex * sel, axis=1, keepdims=True)
    return lax.rsqrt(jnp.clip(d_col, 1.0, None))


def _mm1_body(x_ref, w_ref, deg_ref, o_ref):
    n_out = _norms(deg_ref, RB, 0)
    o_ref[...] = jnp.dot(x_ref[...], w_ref[...],
                         preferred_element_type=jnp.float32) * n_out


def _layer2_body(p_ref, deg_ref, b1_ref, w2_ref, o_ref):
    n_out = _norms(deg_ref, RB, 0)
    n_in = _norms(deg_ref, RB, 1)
    h = jnp.maximum((p_ref[0] + p_ref[1]) * n_in + b1_ref[...], 0.0)
    o_ref[...] = jnp.dot(h, w2_ref[...],
                         preferred_element_type=jnp.float32) * n_out


def _final_body(p_ref, deg_ref, b2_ref, o_ref):
    n_in = _norms(deg_ref, RB, 1)
    o_ref[...] = (p_ref[0] + p_ref[1]) * n_in + b2_ref[...]


def _specs(rb):
    return dict(
        deg=pl.BlockSpec((2, 2, rb // 8, D), lambda i: (0, 0, i, 0)),
        row=pl.BlockSpec((rb, D), lambda i: (i, 0)),
        pair=pl.BlockSpec((2, rb, D), lambda i: (0, i, 0)),
        w=pl.BlockSpec((D, D), lambda i: (0, 0)),
        b=pl.BlockSpec((1, D), lambda i: (0, 0)),
    )


_S = _specs(RB)

_mm1 = pl.pallas_call(
    _mm1_body,
    grid=(GRID,),
    in_specs=[_S["row"], _S["w"], _S["deg"]],
    out_specs=_S["row"],
    out_shape=jax.ShapeDtypeStruct((N_PAD, D), jnp.float32),
)

_layer2 = pl.pallas_call(
    _layer2_body,
    grid=(GRID,),
    in_specs=[_S["pair"], _S["deg"], _S["b"], _S["w"]],
    out_specs=_S["row"],
    out_shape=jax.ShapeDtypeStruct((N_PAD, D), jnp.float32),
)

_final = pl.pallas_call(
    _final_body,
    grid=(GRID,),
    in_specs=[_S["pair"], _S["deg"], _S["b"]],
    out_specs=_S["row"],
    out_shape=jax.ShapeDtypeStruct((N_PAD, D), jnp.float32),
)


def kernel(features, edge_index, W1, b1, W2, b2):
    ei = edge_index.reshape(2, E // 128, 128)
    ei = jnp.pad(ei, ((0, 0), (0, (E_PAD - E) // 128), (0, 0)),
                 constant_values=PAD_NODE)
    ei = ei.reshape(2, EDGE_ROWS, B)
    src = ei[0]
    dst = ei[1]

    xp = jnp.pad(features, ((0, N_PAD - N), (0, 0)))
    b1r = b1.reshape(1, D)
    b2r = b2.reshape(1, D)

    degs = _deg_kernel(src, dst)
    degs_p = degs.reshape(NC, 2, N_PAD // 8, D)

    h1 = _mm1(xp, W1, degs_p)
    p1 = _agg_kernel(h1, src, dst)
    h2 = _layer2(p1, degs_p, b1r, W2)
    p2 = _agg_kernel(h2, src, dst)
    return _final(p2, degs_p, b2r)[:N]


# speedup vs baseline: 1.7870x; 1.7870x over previous
import jax
import jax.numpy as jnp
from jax import lax
from jax.experimental import pallas as pl
from jax.experimental.pallas import tpu as pltpu
from jax.experimental.pallas import tpu_sc as plsc

N = 10000
E = 320000
D = 128

NC = 2
NS = 16

N_PAD = 10240
NPT = N_PAD // NS
PAD_NODE = N_PAD - 1

B = 112
E_PAD = 322560
EDGE_ROWS = E_PAD // B
RPT = EDGE_ROWS // (NC * NS)
NBLK = 6
BLK = RPT // NBLK

ZR_DEG = 64
ZC_DEG = NPT // ZR_DEG

_MESH = plsc.VectorSubcoreMesh(core_axis_name="c", subcore_axis_name="s")
_SC_PARAMS = pltpu.CompilerParams(use_tc_tiling_on_sc=False)


def _zero_vmem(ref, nrows, ncols, dtype=jnp.float32):
    zeros16 = jnp.zeros((16,), dtype)

    def body(i, _):
        for col in range(ncols // 16):
            ref[i, pl.ds(col * 16, 16)] = zeros16
        return 0

    lax.fori_loop(0, nrows, body, 0)


def _deg_body(src_hbm, dst_hbm, out_hbm,
              deg_s, deg_d, src_v, dst_v, ones_v, zbuf, sem_a, sem_b):
    c = lax.axis_index("c")
    s = lax.axis_index("s")

    _zero_vmem(zbuf, ZR_DEG, 16)
    for k in range(ZC_DEG):
        base = s * NPT + k * ZR_DEG
        pltpu.async_copy(zbuf, deg_s.at[pl.ds(base, ZR_DEG)], sem_a)
        pltpu.async_copy(zbuf, deg_d.at[pl.ds(base, ZR_DEG)], sem_b)

    ones16 = jnp.ones((16,), jnp.float32)

    def ones_body(i, _):
        ones_v[i, :] = ones16
        return 0

    lax.fori_loop(0, B, ones_body, 0)

    row0 = c * (EDGE_ROWS // NC) + s * RPT
    pltpu.sync_copy(src_hbm.at[pl.ds(row0, RPT)], src_v)
    pltpu.sync_copy(dst_hbm.at[pl.ds(row0, RPT)], dst_v)

    for k in range(ZC_DEG):
        base = s * NPT + k * ZR_DEG
        pltpu.make_async_copy(zbuf, deg_s.at[pl.ds(base, ZR_DEG)], sem_a).wait()
        pltpu.make_async_copy(zbuf, deg_d.at[pl.ds(base, ZR_DEG)], sem_b).wait()

    plsc.subcore_barrier()

    def chunk(j, _):
        pltpu.async_copy(ones_v, deg_s.at[src_v.at[j]], sem_a, add=True)
        pltpu.async_copy(ones_v, deg_d.at[dst_v.at[j]], sem_b, add=True)

        @pl.when(j >= 4)
        def _():
            pltpu.make_async_copy(ones_v, deg_s.at[src_v.at[j - 4]], sem_a).wait()
            pltpu.make_async_copy(ones_v, deg_d.at[dst_v.at[j - 4]], sem_b).wait()

        return 0

    lax.fori_loop(0, RPT, chunk, 0)
    for j in range(RPT - 4, RPT):
        pltpu.make_async_copy(ones_v, deg_s.at[src_v.at[j]], sem_a).wait()
        pltpu.make_async_copy(ones_v, deg_d.at[dst_v.at[j]], sem_b).wait()

    plsc.subcore_barrier()

    base = s * NPT
    pltpu.sync_copy(deg_s.at[pl.ds(base, NPT)], out_hbm.at[c, 0, pl.ds(base, NPT)])
    pltpu.sync_copy(deg_d.at[pl.ds(base, NPT)], out_hbm.at[c, 1, pl.ds(base, NPT)])


_deg_kernel = pl.kernel(
    _deg_body,
    out_type=jax.ShapeDtypeStruct((NC, 2, N_PAD, 16), jnp.float32),
    mesh=_MESH,
    compiler_params=_SC_PARAMS,
    scratch_types=[
        pltpu.VMEM_SHARED((N_PAD, 16), jnp.float32),
        pltpu.VMEM_SHARED((N_PAD, 16), jnp.float32),
        pltpu.VMEM((RPT, B), jnp.int32),
        pltpu.VMEM((RPT, B), jnp.int32),
        pltpu.VMEM((B, 16), jnp.float32),
        pltpu.VMEM((ZR_DEG, 16), jnp.float32),
        pltpu.SemaphoreType.DMA,
        pltpu.SemaphoreType.DMA,
    ],
)


def _agg_body(h_hbm, src_hbm, dst_hbm, out_hbm,
              acc, src_v, dst_v, rows0, rows1, rows2,
              g0, g1, g2, s0, s1, s2):
    c = lax.axis_index("c")
    s = lax.axis_index("s")

    rows = (rows0, rows1, rows2)
    gsem = (g0, g1, g2)
    ssem = (s0, s1, s2)

    _zero_vmem(rows0, B, D)
    base = s * NPT
    for k in range(5):
        pltpu.async_copy(rows0, acc.at[pl.ds(base + k * B, B)], g0)
    pltpu.async_copy(rows0.at[pl.ds(0, NPT - 5 * B)],
                     acc.at[pl.ds(base + 5 * B, NPT - 5 * B)], g1)

    row0 = c * (EDGE_ROWS // NC) + s * RPT
    pltpu.sync_copy(src_hbm.at[pl.ds(row0, BLK)], src_v)
    pltpu.sync_copy(dst_hbm.at[pl.ds(row0, BLK)], dst_v)

    for k in range(5):
        pltpu.make_async_copy(rows0, acc.at[pl.ds(base + k * B, B)], g0).wait()
    pltpu.make_async_copy(rows0.at[pl.ds(0, NPT - 5 * B)],
                          acc.at[pl.ds(base + 5 * B, NPT - 5 * B)], g1).wait()

    plsc.subcore_barrier()

    for blk in range(NBLK):
        if blk > 0:
            brow = row0 + blk * BLK
            pltpu.sync_copy(src_hbm.at[pl.ds(brow, BLK)], src_v)
            pltpu.sync_copy(dst_hbm.at[pl.ds(brow, BLK)], dst_v)

        pltpu.async_copy(h_hbm.at[src_v.at[0]], rows0, g0)
        pltpu.async_copy(h_hbm.at[src_v.at[1]], rows1, g1)

        def slot(j, b, bp):
            pltpu.make_async_copy(h_hbm.at[src_v.at[j]], rows[b], gsem[b]).wait()
            pltpu.async_copy(rows[b], acc.at[dst_v.at[j]], ssem[b], add=True)

            @pl.when(j >= 1)
            def _():
                pltpu.make_async_copy(rows[bp], acc.at[dst_v.at[j - 1]],
                                      ssem[bp]).wait()

            @pl.when(j + 2 < BLK)
            def _():
                pltpu.async_copy(h_hbm.at[src_v.at[j + 2]], rows[bp], gsem[bp])

        def tri(k, _):
            for i in range(3):
                slot(k * 3 + i, i, (i + 2) % 3)
            return 0

        lax.fori_loop(0, BLK // 3, tri, 0)
        bl = BLK - 1
        pltpu.make_async_copy(rows[bl % 3], acc.at[dst_v.at[bl]],
                              ssem[bl % 3]).wait()

    plsc.subcore_barrier()

    pltpu.sync_copy(acc.at[pl.ds(base, NPT)], out_hbm.at[c, pl.ds(base, NPT)])


_agg_kernel = pl.kernel(
    _agg_body,
    out_type=jax.ShapeDtypeStruct((NC, N_PAD, D), jnp.float32),
    mesh=_MESH,
    compiler_params=_SC_PARAMS,
    scratch_types=[
        pltpu.VMEM_SHARED((N_PAD, D), jnp.float32),
        pltpu.VMEM((BLK, B), jnp.int32),
        pltpu.VMEM((BLK, B), jnp.int32),
        pltpu.VMEM((B, D), jnp.float32),
        pltpu.VMEM((B, D), jnp.float32),
        pltpu.VMEM((B, D), jnp.float32),
        pltpu.SemaphoreType.DMA,
        pltpu.SemaphoreType.DMA,
        pltpu.SemaphoreType.DMA,
        pltpu.SemaphoreType.DMA,
        pltpu.SemaphoreType.DMA,
        pltpu.SemaphoreType.DMA,
    ],
)


RB = 512
GRID = N_PAD // RB


def _norms(deg_ref, rb, which):
    dp = deg_ref[...]
    d_p = dp[0, which] + dp[1, which]
    rq = lax.broadcasted_iota(jnp.int32, (rb, rb // 8), 0) // 8
    qq = lax.broadcasted_iota(jnp.int32, (rb, rb // 8), 1)
    a = (rq == qq).astype(jnp.float32)
    ex = jnp.dot(a, d_p, preferred_element_type=jnp.float32)
    rr = lax.broadcasted_iota(jnp.int32, (rb, 128), 0) % 8
    ll = lax.broadcasted_iota(jnp.int32, (rb, 128), 1)
    sel = (ll == 16 * rr).astype(jnp.float32)
    d_col = jnp.sum(ex * sel, axis=1, keepdims=True)
    return lax.rsqrt(jnp.clip(d_col, 1.0, None))


def _mm1_body(x_ref, w_ref, deg_ref, o_ref):
    n_out = _norms(deg_ref, RB, 0)
    o_ref[...] = jnp.dot(x_ref[...], w_ref[...],
                         preferred_element_type=jnp.float32) * n_out


def _layer2_body(p_ref, deg_ref, b1_ref, w2_ref, o_ref):
    n_out = _norms(deg_ref, RB, 0)
    n_in = _norms(deg_ref, RB, 1)
    h = jnp.maximum((p_ref[0] + p_ref[1]) * n_in + b1_ref[...], 0.0)
    o_ref[...] = jnp.dot(h, w2_ref[...],
                         preferred_element_type=jnp.float32) * n_out


def _final_body(p_ref, deg_ref, b2_ref, o_ref):
    n_in = _norms(deg_ref, RB, 1)
    o_ref[...] = (p_ref[0] + p_ref[1]) * n_in + b2_ref[...]


def _specs(rb):
    return dict(
        deg=pl.BlockSpec((2, 2, rb // 8, D), lambda i: (0, 0, i, 0)),
        row=pl.BlockSpec((rb, D), lambda i: (i, 0)),
        pair=pl.BlockSpec((2, rb, D), lambda i: (0, i, 0)),
        w=pl.BlockSpec((D, D), lambda i: (0, 0)),
        b=pl.BlockSpec((1, D), lambda i: (0, 0)),
    )


_S = _specs(RB)

_mm1 = pl.pallas_call(
    _mm1_body,
    grid=(GRID,),
    in_specs=[_S["row"], _S["w"], _S["deg"]],
    out_specs=_S["row"],
    out_shape=jax.ShapeDtypeStruct((N_PAD, D), jnp.float32),
)

_layer2 = pl.pallas_call(
    _layer2_body,
    grid=(GRID,),
    in_specs=[_S["pair"], _S["deg"], _S["b"], _S["w"]],
    out_specs=_S["row"],
    out_shape=jax.ShapeDtypeStruct((N_PAD, D), jnp.float32),
)

_final = pl.pallas_call(
    _final_body,
    grid=(GRID,),
    in_specs=[_S["pair"], _S["deg"], _S["b"]],
    out_specs=_S["row"],
    out_shape=jax.ShapeDtypeStruct((N_PAD, D), jnp.float32),
)


def kernel(features, edge_index, W1, b1, W2, b2):
    ei = edge_index.reshape(2, E // 128, 128)
    npadrows = (E_PAD - E) // 128
    fill = N + (jnp.arange(npadrows * 128, dtype=jnp.int32) % (N_PAD - N))
    fill = jnp.broadcast_to(fill.reshape(1, npadrows, 128), (2, npadrows, 128))
    ei = jnp.concatenate([ei, fill], axis=1)
    ei = ei.reshape(2, EDGE_ROWS, B)
    src = ei[0]
    dst = ei[1]

    xp = jnp.pad(features, ((0, N_PAD - N), (0, 0)))
    b1r = b1.reshape(1, D)
    b2r = b2.reshape(1, D)

    degs = _deg_kernel(src, dst)
    degs_p = degs.reshape(NC, 2, N_PAD // 8, D)

    h1 = _mm1(xp, W1, degs_p)
    p1 = _agg_kernel(h1, src, dst)
    h2 = _layer2(p1, degs_p, b1r, W2)
    p2 = _agg_kernel(h2, src, dst)
    return _final(p2, degs_p, b2r)[:N]


# speedup vs baseline: 1.9060x; 1.0666x over previous
import jax
import jax.numpy as jnp
from jax import lax
from jax.experimental import pallas as pl
from jax.experimental.pallas import tpu as pltpu
from jax.experimental.pallas import tpu_sc as plsc

N = 10000
E = 320000
D = 128

NC = 2
NS = 16

N_PAD = 10240
NPT = N_PAD // NS
PAD_NODE = N_PAD - 1

B = 112
E_PAD = 322560
EDGE_ROWS = E_PAD // B
RPT = EDGE_ROWS // (NC * NS)
NBLK = 5
BLK = RPT // NBLK

ZR_DEG = 64
ZC_DEG = NPT // ZR_DEG

_MESH = plsc.VectorSubcoreMesh(core_axis_name="c", subcore_axis_name="s")
_SC_PARAMS = pltpu.CompilerParams(use_tc_tiling_on_sc=False)


def _zero_vmem(ref, nrows, ncols, dtype=jnp.float32):
    zeros16 = jnp.zeros((16,), dtype)

    def body(i, _):
        for col in range(ncols // 16):
            ref[i, pl.ds(col * 16, 16)] = zeros16
        return 0

    lax.fori_loop(0, nrows, body, 0)


def _deg_body(src_hbm, dst_hbm, out_hbm,
              deg_s, deg_d, src_v, dst_v, ones_v, zbuf, sem_a, sem_b):
    c = lax.axis_index("c")
    s = lax.axis_index("s")

    _zero_vmem(zbuf, ZR_DEG, 16)
    for k in range(ZC_DEG):
        base = s * NPT + k * ZR_DEG
        pltpu.async_copy(zbuf, deg_s.at[pl.ds(base, ZR_DEG)], sem_a)
        pltpu.async_copy(zbuf, deg_d.at[pl.ds(base, ZR_DEG)], sem_b)

    ones16 = jnp.ones((16,), jnp.float32)

    def ones_body(i, _):
        ones_v[i, :] = ones16
        return 0

    lax.fori_loop(0, B, ones_body, 0)

    row0 = c * (EDGE_ROWS // NC) + s * RPT
    pltpu.sync_copy(src_hbm.at[pl.ds(row0, RPT)], src_v)
    pltpu.sync_copy(dst_hbm.at[pl.ds(row0, RPT)], dst_v)

    for k in range(ZC_DEG):
        base = s * NPT + k * ZR_DEG
        pltpu.make_async_copy(zbuf, deg_s.at[pl.ds(base, ZR_DEG)], sem_a).wait()
        pltpu.make_async_copy(zbuf, deg_d.at[pl.ds(base, ZR_DEG)], sem_b).wait()

    plsc.subcore_barrier()

    def chunk(j, _):
        pltpu.async_copy(ones_v, deg_s.at[src_v.at[j]], sem_a, add=True)
        pltpu.async_copy(ones_v, deg_d.at[dst_v.at[j]], sem_b, add=True)

        @pl.when(j >= 8)
        def _():
            pltpu.make_async_copy(ones_v, deg_s.at[src_v.at[j - 8]], sem_a).wait()
            pltpu.make_async_copy(ones_v, deg_d.at[dst_v.at[j - 8]], sem_b).wait()

        return 0

    lax.fori_loop(0, RPT, chunk, 0)
    for j in range(RPT - 8, RPT):
        pltpu.make_async_copy(ones_v, deg_s.at[src_v.at[j]], sem_a).wait()
        pltpu.make_async_copy(ones_v, deg_d.at[dst_v.at[j]], sem_b).wait()

    plsc.subcore_barrier()

    base = s * NPT
    pltpu.sync_copy(deg_s.at[pl.ds(base, NPT)], out_hbm.at[c, 0, pl.ds(base, NPT)])
    pltpu.sync_copy(deg_d.at[pl.ds(base, NPT)], out_hbm.at[c, 1, pl.ds(base, NPT)])


_deg_kernel = pl.kernel(
    _deg_body,
    out_type=jax.ShapeDtypeStruct((NC, 2, N_PAD, 16), jnp.float32),
    mesh=_MESH,
    compiler_params=_SC_PARAMS,
    scratch_types=[
        pltpu.VMEM_SHARED((N_PAD, 16), jnp.float32),
        pltpu.VMEM_SHARED((N_PAD, 16), jnp.float32),
        pltpu.VMEM((RPT, B), jnp.int32),
        pltpu.VMEM((RPT, B), jnp.int32),
        pltpu.VMEM((B, 16), jnp.float32),
        pltpu.VMEM((ZR_DEG, 16), jnp.float32),
        pltpu.SemaphoreType.DMA,
        pltpu.SemaphoreType.DMA,
    ],
)


def _agg_body(h_hbm, src_hbm, dst_hbm, out_hbm,
              acc, src_v, dst_v, rows0, rows1, rows2,
              g0, g1, g2, s0, s1, s2):
    c = lax.axis_index("c")
    s = lax.axis_index("s")

    rows = (rows0, rows1, rows2)
    gsem = (g0, g1, g2)
    ssem = (s0, s1, s2)

    _zero_vmem(rows0, B, D)
    base = s * NPT
    for k in range(5):
        pltpu.async_copy(rows0, acc.at[pl.ds(base + k * B, B)], g0)
    pltpu.async_copy(rows0.at[pl.ds(0, NPT - 5 * B)],
                     acc.at[pl.ds(base + 5 * B, NPT - 5 * B)], g1)

    row0 = c * (EDGE_ROWS // NC) + s * RPT
    pltpu.sync_copy(src_hbm.at[pl.ds(row0, BLK)], src_v)
    pltpu.sync_copy(dst_hbm.at[pl.ds(row0, BLK)], dst_v)

    for k in range(5):
        pltpu.make_async_copy(rows0, acc.at[pl.ds(base + k * B, B)], g0).wait()
    pltpu.make_async_copy(rows0.at[pl.ds(0, NPT - 5 * B)],
                          acc.at[pl.ds(base + 5 * B, NPT - 5 * B)], g1).wait()

    plsc.subcore_barrier()

    for blk in range(NBLK):
        if blk > 0:
            brow = row0 + blk * BLK
            pltpu.sync_copy(src_hbm.at[pl.ds(brow, BLK)], src_v)
            pltpu.sync_copy(dst_hbm.at[pl.ds(brow, BLK)], dst_v)

        pltpu.async_copy(h_hbm.at[src_v.at[0]], rows0, g0)
        pltpu.async_copy(h_hbm.at[src_v.at[1]], rows1, g1)

        def slot(j, b, bp):
            pltpu.make_async_copy(h_hbm.at[src_v.at[j]], rows[b], gsem[b]).wait()
            pltpu.async_copy(rows[b], acc.at[dst_v.at[j]], ssem[b], add=True)

            @pl.when(j >= 1)
            def _():
                pltpu.make_async_copy(rows[bp], acc.at[dst_v.at[j - 1]],
                                      ssem[bp]).wait()

            @pl.when(j + 2 < BLK)
            def _():
                pltpu.async_copy(h_hbm.at[src_v.at[j + 2]], rows[bp], gsem[bp])

        def tri(k, _):
            for i in range(3):
                slot(k * 3 + i, i, (i + 2) % 3)
            return 0

        lax.fori_loop(0, BLK // 3, tri, 0)
        bl = BLK - 1
        pltpu.make_async_copy(rows[bl % 3], acc.at[dst_v.at[bl]],
                              ssem[bl % 3]).wait()

    plsc.subcore_barrier()

    pltpu.sync_copy(acc.at[pl.ds(base, NPT)], out_hbm.at[c, pl.ds(base, NPT)])


_agg_kernel = pl.kernel(
    _agg_body,
    out_type=jax.ShapeDtypeStruct((NC, N_PAD, D), jnp.float32),
    mesh=_MESH,
    compiler_params=_SC_PARAMS,
    scratch_types=[
        pltpu.VMEM_SHARED((N_PAD, D), jnp.float32),
        pltpu.VMEM((BLK, B), jnp.int32),
        pltpu.VMEM((BLK, B), jnp.int32),
        pltpu.VMEM((B, D), jnp.float32),
        pltpu.VMEM((B, D), jnp.float32),
        pltpu.VMEM((B, D), jnp.float32),
        pltpu.SemaphoreType.DMA,
        pltpu.SemaphoreType.DMA,
        pltpu.SemaphoreType.DMA,
        pltpu.SemaphoreType.DMA,
        pltpu.SemaphoreType.DMA,
        pltpu.SemaphoreType.DMA,
    ],
)


RB = 1024
GRID = N_PAD // RB


def _norms(deg_ref, rb, which):
    dp = deg_ref[...]
    d_p = dp[0, which] + dp[1, which]
    rq = lax.broadcasted_iota(jnp.int32, (rb, rb // 8), 0) // 8
    qq = lax.broadcasted_iota(jnp.int32, (rb, rb // 8), 1)
    a = (rq == qq).astype(jnp.float32)
    ex = jnp.dot(a, d_p, preferred_element_type=jnp.float32)
    rr = lax.broadcasted_iota(jnp.int32, (rb, 128), 0) % 8
    ll = lax.broadcasted_iota(jnp.int32, (rb, 128), 1)
    sel = (ll == 16 * rr).astype(jnp.float32)
    d_col = jnp.sum(ex * sel, axis=1, keepdims=True)
    return lax.rsqrt(jnp.clip(d_col, 1.0, None))


def _mm1_body(x_ref, w_ref, deg_ref, o_ref):
    n_out = _norms(deg_ref, RB, 0)
    o_ref[...] = jnp.dot(x_ref[...], w_ref[...],
                         preferred_element_type=jnp.float32) * n_out


def _layer2_body(p_ref, deg_ref, b1_ref, w2_ref, o_ref):
    n_out = _norms(deg_ref, RB, 0)
    n_in = _norms(deg_ref, RB, 1)
    h = jnp.maximum((p_ref[0] + p_ref[1]) * n_in + b1_ref[...], 0.0)
    o_ref[...] = jnp.dot(h, w2_ref[...],
                         preferred_element_type=jnp.float32) * n_out


def _final_body(p_ref, deg_ref, b2_ref, o_ref):
    n_in = _norms(deg_ref, RB, 1)
    o_ref[...] = (p_ref[0] + p_ref[1]) * n_in + b2_ref[...]


def _specs(rb):
    return dict(
        deg=pl.BlockSpec((2, 2, rb // 8, D), lambda i: (0, 0, i, 0)),
        row=pl.BlockSpec((rb, D), lambda i: (i, 0)),
        pair=pl.BlockSpec((2, rb, D), lambda i: (0, i, 0)),
        w=pl.BlockSpec((D, D), lambda i: (0, 0)),
        b=pl.BlockSpec((1, D), lambda i: (0, 0)),
    )


_S = _specs(RB)

_mm1 = pl.pallas_call(
    _mm1_body,
    grid=(GRID,),
    in_specs=[_S["row"], _S["w"], _S["deg"]],
    out_specs=_S["row"],
    out_shape=jax.ShapeDtypeStruct((N_PAD, D), jnp.float32),
)

_layer2 = pl.pallas_call(
    _layer2_body,
    grid=(GRID,),
    in_specs=[_S["pair"], _S["deg"], _S["b"], _S["w"]],
    out_specs=_S["row"],
    out_shape=jax.ShapeDtypeStruct((N_PAD, D), jnp.float32),
)

_final = pl.pallas_call(
    _final_body,
    grid=(GRID,),
    in_specs=[_S["pair"], _S["deg"], _S["b"]],
    out_specs=_S["row"],
    out_shape=jax.ShapeDtypeStruct((N_PAD, D), jnp.float32),
)


def kernel(features, edge_index, W1, b1, W2, b2):
    ei = edge_index.reshape(2, E // 128, 128)
    npadrows = (E_PAD - E) // 128
    fill = N + (jnp.arange(npadrows * 128, dtype=jnp.int32) % (N_PAD - N))
    fill = jnp.broadcast_to(fill.reshape(1, npadrows, 128), (2, npadrows, 128))
    ei = jnp.concatenate([ei, fill], axis=1)
    ei = ei.reshape(2, EDGE_ROWS, B)
    src = ei[0]
    dst = ei[1]

    xp = jnp.pad(features, ((0, N_PAD - N), (0, 0)))
    b1r = b1.reshape(1, D)
    b2r = b2.reshape(1, D)

    degs = _deg_kernel(src, dst)
    degs_p = degs.reshape(NC, 2, N_PAD // 8, D)

    h1 = _mm1(xp, W1, degs_p)
    p1 = _agg_kernel(h1, src, dst)
    h2 = _layer2(p1, degs_p, b1r, W2)
    p2 = _agg_kernel(h2, src, dst)
    return _final(p2, degs_p, b2r)[:N]
